# baseline (speedup 1.0000x reference)
import functools

import jax
import jax.numpy as jnp
from jax import lax
from jax.experimental import pallas as pl
from jax.experimental.pallas import tpu as pltpu
from jax.experimental.pallas import tpu_sc as plsc

B, C, H, W = 32, 768, 24, 24
HW = H * W


def _sc_gather(emb_table, idx):
    mesh = plsc.VectorSubcoreMesh(core_axis_name="c", subcore_axis_name="s")

    @functools.partial(
        pl.kernel,
        mesh=mesh,
        out_type=jax.ShapeDtypeStruct((B, C), jnp.float32),
        scratch_types=[
            pltpu.VMEM((B,), jnp.int32),
            pltpu.VMEM((B, C), jnp.float32),
            pltpu.SemaphoreType.DMA,
        ],
    )
    def gather_kernel(table_hbm, idx_hbm, out_hbm, idx_v, rows_v, sem):
        wid = lax.axis_index("s") * 2 + lax.axis_index("c")

        @pl.when(wid == 0)
        def _():
            pltpu.sync_copy(idx_hbm, idx_v)
            pltpu.async_copy(table_hbm.at[idx_v], rows_v, sem).wait()
            pltpu.sync_copy(rows_v, out_hbm)

    return gather_kernel(emb_table, idx)


def _add_body(x_ref, e_ref, o_ref):
    o_ref[...] = x_ref[...] + e_ref[...]


def _tc_add(x3, e3):
    return pl.pallas_call(
        _add_body,
        grid=(B,),
        in_specs=[
            pl.BlockSpec((1, C, HW), lambda b: (b, 0, 0)),
            pl.BlockSpec((1, C, 1), lambda b: (b, 0, 0)),
        ],
        out_specs=pl.BlockSpec((1, C, HW), lambda b: (b, 0, 0)),
        out_shape=jax.ShapeDtypeStruct((B, C, HW), jnp.float32),
    )(x3, e3)


def kernel(x, anatomy_idx, emb_table):
    e = _sc_gather(emb_table, anatomy_idx.astype(jnp.int32))
    out = _tc_add(x.reshape(B, C, HW), e[:, :, None])
    return out.reshape(B, C, H, W)


# speedup vs baseline: 1.1321x; 1.1321x over previous
import jax
import jax.numpy as jnp
from jax.experimental import pallas as pl
from jax.experimental.pallas import tpu as pltpu

B, C, H, W = 32, 768, 24, 24
HW = H * W


def _body(idx_ref, x_ref, e_ref, o_ref):
    o_ref[...] = x_ref[...] + e_ref[...]


def kernel(x, anatomy_idx, emb_table):
    x3 = x.reshape(B, C, HW)
    emb3 = emb_table[:, :, None]
    out = pl.pallas_call(
        _body,
        grid_spec=pltpu.PrefetchScalarGridSpec(
            num_scalar_prefetch=1,
            grid=(B,),
            in_specs=[
                pl.BlockSpec((1, C, HW), lambda b, idx: (b, 0, 0)),
                pl.BlockSpec((1, C, 1), lambda b, idx: (idx[b], 0, 0)),
            ],
            out_specs=pl.BlockSpec((1, C, HW), lambda b, idx: (b, 0, 0)),
        ),
        out_shape=jax.ShapeDtypeStruct((B, C, HW), jnp.float32),
    )(anatomy_idx.astype(jnp.int32), x3, emb3)
    return out.reshape(B, C, H, W)
